# baseline (speedup 1.0000x reference)
import functools
import math

import jax
import jax.numpy as jnp
from jax import lax
from jax.experimental import pallas as pl
from jax.experimental.pallas import tpu as pltpu
from jax.experimental.pallas import tpu_sc as plsc

VOCAB = 1000000
HDIM = 64
BATCH = 4096
SEQLEN = 200

NUM_WORKERS = 32
ROWS = BATCH * SEQLEN
ROWS_PER_WORKER = ROWS // NUM_WORKERS
CHUNK = SEQLEN
NCHUNK = ROWS_PER_WORKER // CHUNK

GATHER_SPLITS = ((0, 104), (104, 96))

TBLOCK = 4096
BBLOCK = 128


def _sinusoidal_pe(length, d_model):
    pos = jnp.arange(length, dtype=jnp.float32)[:, None]
    i = jnp.arange(0, d_model, 2, dtype=jnp.float32)
    div = jnp.exp(-(math.log(10000.0)) * i / d_model)
    pe = jnp.zeros((length, d_model), dtype=jnp.float32)
    pe = pe.at[:, 0::2].set(jnp.sin(pos * div))
    pe = pe.at[:, 1::2].set(jnp.cos(pos * div))
    return pe


def _transpose_table(table_t):
    def body(in_ref, out_ref):
        out_ref[...] = in_ref[...].T

    grid = (VOCAB + TBLOCK - 1) // TBLOCK
    return pl.pallas_call(
        body,
        grid=(grid,),
        in_specs=[pl.BlockSpec((HDIM, TBLOCK), lambda i: (0, i))],
        out_specs=pl.BlockSpec((TBLOCK, HDIM), lambda i: (i, 0)),
        out_shape=jax.ShapeDtypeStruct((VOCAB, HDIM), jnp.float32),
    )(table_t)


LB = 8


def _transpose_out(emb3):
    def body(in_ref, out_ref):
        for j in range(LB):
            out_ref[j] = in_ref[:, j, :].T

    return pl.pallas_call(
        body,
        grid=(SEQLEN // LB, BATCH // BBLOCK),
        in_specs=[pl.BlockSpec((BBLOCK, LB, HDIM), lambda l, b: (b, l, 0))],
        out_specs=pl.BlockSpec((LB, HDIM, BBLOCK), lambda l, b: (l, 0, b)),
        out_shape=jax.ShapeDtypeStruct((SEQLEN, HDIM, BATCH), jnp.float32),
    )(emb3)


def _make_sc_kernel():
    mesh = plsc.VectorSubcoreMesh(core_axis_name="c", subcore_axis_name="s",
                                  num_cores=2, num_subcores=16)

    @functools.partial(
        pl.kernel,
        mesh=mesh,
        out_type=jax.ShapeDtypeStruct((ROWS, HDIM), jnp.float32),
        scratch_types=[
            pltpu.VMEM((ROWS_PER_WORKER,), jnp.int32),
            pltpu.VMEM((CHUNK, HDIM), jnp.float32),
            pltpu.VMEM((CHUNK, HDIM), jnp.float32),
            pltpu.VMEM((SEQLEN, HDIM), jnp.float32),
            pltpu.SemaphoreType.DMA,
            pltpu.SemaphoreType.DMA,
        ],
        compiler_params=pltpu.CompilerParams(use_tc_tiling_on_sc=False),
    )
    def k(idx_hbm, pe_hbm, table_hbm, out_hbm,
          idx_v, buf0, buf1, pe_v, gsem0, gsem1):
        wid = lax.axis_index("s") * 2 + lax.axis_index("c")
        base = wid * ROWS_PER_WORKER
        bufs = (buf0, buf1)
        gsems = (gsem0, gsem1)

        pltpu.sync_copy(pe_hbm, pe_v)
        pltpu.sync_copy(idx_hbm.at[pl.ds(base, ROWS_PER_WORKER)], idx_v)

        def issue_gathers(chunk_i, b):
            for off, width in GATHER_SPLITS:
                pltpu.async_copy(
                    table_hbm.at[idx_v.at[pl.ds(chunk_i * CHUNK + off,
                                                width)]],
                    bufs[b].at[pl.ds(off, width)],
                    gsems[b],
                )

        def finish_chunk(chunk_i, b):
            for off, width in GATHER_SPLITS:
                pltpu.make_async_copy(
                    table_hbm.at[idx_v.at[pl.ds(off, width)]],
                    bufs[b].at[pl.ds(off, width)],
                    gsems[b],
                ).wait()

            def pe_body(r):
                for c in range(HDIM // 16):
                    plsc.addupdate(bufs[b].at[r, pl.ds(c * 16, 16)],
                                   pe_v[r, pl.ds(c * 16, 16)])
            pl.loop(0, CHUNK, unroll=4)(pe_body)

            pltpu.sync_copy(bufs[b],
                            out_hbm.at[pl.ds(base + chunk_i * CHUNK, CHUNK)])

        issue_gathers(0, 0)

        def body(ii):
            for b in range(2):
                chunk_i = ii + b
                issue_gathers(chunk_i + 1, 1 - b)
                finish_chunk(chunk_i, b)
        pl.loop(0, NCHUNK - 2, step=2)(body)

        issue_gathers(NCHUNK - 1, 1)
        finish_chunk(NCHUNK - 2, 0)
        finish_chunk(NCHUNK - 1, 1)

    return k


def kernel(text, table):
    idx = text.reshape(ROWS).astype(jnp.int32)
    pe = _sinusoidal_pe(SEQLEN, HDIM)
    table_rows = _transpose_table(table.T)
    emb = _make_sc_kernel()(idx, pe, table_rows)
    out_t = _transpose_out(emb.reshape(BATCH, SEQLEN, HDIM))
    return jnp.transpose(out_t, (2, 0, 1))


# speedup vs baseline: 1.3253x; 1.3253x over previous
import functools
import math

import jax
import jax.numpy as jnp
from jax import lax
from jax.experimental import pallas as pl
from jax.experimental.pallas import tpu as pltpu
from jax.experimental.pallas import tpu_sc as plsc

VOCAB = 1000000
HDIM = 64
BATCH = 4096
SEQLEN = 200

NUM_WORKERS = 32
ROWS = BATCH * SEQLEN
ROWS_PER_WORKER = ROWS // NUM_WORKERS
CHUNK = SEQLEN
NCHUNK = ROWS_PER_WORKER // CHUNK

GATHER_SPLITS = ((0, 104), (104, 96))

TBLOCK = 4096
BBLOCK = 128


def _sinusoidal_pe(length, d_model):
    pos = jnp.arange(length, dtype=jnp.float32)[:, None]
    i = jnp.arange(0, d_model, 2, dtype=jnp.float32)
    div = jnp.exp(-(math.log(10000.0)) * i / d_model)
    pe = jnp.zeros((length, d_model), dtype=jnp.float32)
    pe = pe.at[:, 0::2].set(jnp.sin(pos * div))
    pe = pe.at[:, 1::2].set(jnp.cos(pos * div))
    return pe


def _transpose_table(table_t):
    def body(in_ref, out_ref):
        out_ref[...] = in_ref[...].T

    grid = (VOCAB + TBLOCK - 1) // TBLOCK
    return pl.pallas_call(
        body,
        grid=(grid,),
        in_specs=[pl.BlockSpec((HDIM, TBLOCK), lambda i: (0, i))],
        out_specs=pl.BlockSpec((TBLOCK, HDIM), lambda i: (i, 0)),
        out_shape=jax.ShapeDtypeStruct((VOCAB, HDIM), jnp.float32),
    )(table_t)


LB = 8


def _transpose_out(emb3):
    def body(in_ref, out_ref):
        for j in range(LB):
            out_ref[j] = in_ref[:, j, :].T

    return pl.pallas_call(
        body,
        grid=(SEQLEN // LB, BATCH // BBLOCK),
        in_specs=[pl.BlockSpec((BBLOCK, LB, HDIM), lambda l, b: (b, l, 0))],
        out_specs=pl.BlockSpec((LB, HDIM, BBLOCK), lambda l, b: (l, 0, b)),
        out_shape=jax.ShapeDtypeStruct((SEQLEN, HDIM, BATCH), jnp.float32),
    )(emb3)


def _make_sc_kernel():
    mesh = plsc.VectorSubcoreMesh(core_axis_name="c", subcore_axis_name="s",
                                  num_cores=2, num_subcores=16)

    @functools.partial(
        pl.kernel,
        mesh=mesh,
        out_type=jax.ShapeDtypeStruct((ROWS, HDIM), jnp.float32),
        scratch_types=[
            pltpu.VMEM((ROWS_PER_WORKER,), jnp.int32),
            pltpu.VMEM((CHUNK, HDIM), jnp.float32),
            pltpu.VMEM((CHUNK, HDIM), jnp.float32),
            pltpu.VMEM((SEQLEN, HDIM), jnp.float32),
            pltpu.SemaphoreType.DMA,
            pltpu.SemaphoreType.DMA,
        ],
        compiler_params=pltpu.CompilerParams(use_tc_tiling_on_sc=False),
    )
    def k(idx_hbm, pe_hbm, table_hbm, out_hbm,
          idx_v, buf0, buf1, pe_v, gsem0, gsem1):
        wid = lax.axis_index("s") * 2 + lax.axis_index("c")
        base = wid * ROWS_PER_WORKER
        bufs = (buf0, buf1)
        gsems = (gsem0, gsem1)

        pltpu.sync_copy(pe_hbm, pe_v)
        pltpu.sync_copy(idx_hbm.at[pl.ds(base, ROWS_PER_WORKER)], idx_v)

        def issue_gathers(chunk_i, b):
            for off, width in GATHER_SPLITS:
                pltpu.async_copy(
                    table_hbm.at[idx_v.at[pl.ds(chunk_i * CHUNK + off,
                                                width)]],
                    bufs[b].at[pl.ds(off, width)],
                    gsems[b],
                )

        def finish_chunk(chunk_i, b):
            for off, width in GATHER_SPLITS:
                pltpu.make_async_copy(
                    table_hbm.at[idx_v.at[pl.ds(off, width)]],
                    bufs[b].at[pl.ds(off, width)],
                    gsems[b],
                ).wait()

            def pe_body(r):
                for c in range(HDIM // 16):
                    plsc.addupdate(bufs[b].at[r, pl.ds(c * 16, 16)],
                                   pe_v[r, pl.ds(c * 16, 16)])
            pl.loop(0, CHUNK, unroll=4)(pe_body)

            pltpu.sync_copy(bufs[b],
                            out_hbm.at[pl.ds(base + chunk_i * CHUNK, CHUNK)])

        issue_gathers(0, 0)

        def body(ii):
            for b in range(2):
                chunk_i = ii + b
                issue_gathers(chunk_i + 1, 1 - b)
                finish_chunk(chunk_i, b)
        pl.loop(0, NCHUNK - 2, step=2)(body)

        issue_gathers(NCHUNK - 1, 1)
        finish_chunk(NCHUNK - 2, 0)
        finish_chunk(NCHUNK - 1, 1)

    return k


def kernel(text, table):
    idx = text.reshape(ROWS).astype(jnp.int32)
    pe = _sinusoidal_pe(SEQLEN, HDIM)
    table_rows = _transpose_table(table.T)
    emb = _make_sc_kernel()(idx, pe, table_rows)
    return emb.reshape(BATCH, SEQLEN, HDIM)


# speedup vs baseline: 1.3908x; 1.0494x over previous
import functools
import math

import jax
import jax.numpy as jnp
from jax import lax
from jax.experimental import pallas as pl
from jax.experimental.pallas import tpu as pltpu
from jax.experimental.pallas import tpu_sc as plsc

VOCAB = 1000000
HDIM = 64
BATCH = 4096
SEQLEN = 200

NUM_WORKERS = 32
ROWS = BATCH * SEQLEN
ROWS_PER_WORKER = ROWS // NUM_WORKERS
CHUNK = 2 * SEQLEN
NCHUNK = ROWS_PER_WORKER // CHUNK

GATHER_SPLITS = ((0, 104), (104, 96), (200, 104), (304, 96))

BB = 1024
LB = 8


def _sinusoidal_pe(length, d_model):
    pos = jnp.arange(length, dtype=jnp.float32)[:, None]
    i = jnp.arange(0, d_model, 2, dtype=jnp.float32)
    div = jnp.exp(-(math.log(10000.0)) * i / d_model)
    pe = jnp.zeros((length, d_model), dtype=jnp.float32)
    pe = pe.at[:, 0::2].set(jnp.sin(pos * div))
    pe = pe.at[:, 1::2].set(jnp.cos(pos * div))
    return pe


def _transpose_out(emb128):
    emb3 = emb128.reshape(BATCH, SEQLEN, 128)

    def body(in_ref, out_ref):
        for j in range(LB):
            out_ref[j] = in_ref[:, j, 0:HDIM].T

    return pl.pallas_call(
        body,
        grid=(SEQLEN // LB, BATCH // BB),
        in_specs=[pl.BlockSpec((BB, LB, 128), lambda l, b: (b, l, 0))],
        out_specs=pl.BlockSpec((LB, HDIM, BB), lambda l, b: (l, 0, b)),
        out_shape=jax.ShapeDtypeStruct((SEQLEN, HDIM, BATCH), jnp.float32),
    )(emb3)


def _make_sc_kernel():
    mesh = plsc.VectorSubcoreMesh(core_axis_name="c", subcore_axis_name="s",
                                  num_cores=2, num_subcores=16)

    @functools.partial(
        pl.kernel,
        mesh=mesh,
        out_type=jax.ShapeDtypeStruct((ROWS, 128), jnp.float32),
        scratch_types=[
            pltpu.VMEM((CHUNK,), jnp.int32),
            pltpu.VMEM((CHUNK,), jnp.int32),
            pltpu.VMEM((CHUNK, 128), jnp.float32),
            pltpu.VMEM((CHUNK, 128), jnp.float32),
            pltpu.VMEM((SEQLEN, HDIM), jnp.float32),
            pltpu.SemaphoreType.DMA,
            pltpu.SemaphoreType.DMA,
            pltpu.SemaphoreType.DMA,
            pltpu.SemaphoreType.DMA,
        ],
        compiler_params=pltpu.CompilerParams(use_tc_tiling_on_sc=False),
    )
    def k(idx_hbm, pe_hbm, table_hbm, out_hbm,
          idx0, idx1, buf0, buf1, pe_v, isem0, isem1, gsem0, gsem1):
        wid = lax.axis_index("s") * 2 + lax.axis_index("c")
        base = wid * ROWS_PER_WORKER
        idxs = (idx0, idx1)
        bufs = (buf0, buf1)
        isems = (isem0, isem1)
        gsems = (gsem0, gsem1)

        pltpu.sync_copy(pe_hbm, pe_v)

        def issue_idx(chunk_i, b):
            pltpu.async_copy(
                idx_hbm.at[pl.ds(base + chunk_i * CHUNK, CHUNK)],
                idxs[b], isems[b])

        def issue_gathers(chunk_i, b):
            pltpu.make_async_copy(
                idx_hbm.at[pl.ds(0, CHUNK)], idxs[b], isems[b]).wait()
            for off, width in GATHER_SPLITS:
                pltpu.async_copy(
                    table_hbm.at[idxs[b].at[pl.ds(off, width)]],
                    bufs[b].at[pl.ds(off, width)],
                    gsems[b],
                )

        def finish_chunk(chunk_i, b):
            for off, width in GATHER_SPLITS:
                pltpu.make_async_copy(
                    table_hbm.at[idxs[b].at[pl.ds(off, width)]],
                    bufs[b].at[pl.ds(off, width)],
                    gsems[b],
                ).wait()

            def pe_body(r):
                for c in range(HDIM // 16):
                    s = pl.ds(c * 16, 16)
                    plsc.addupdate(bufs[b].at[r, s], pe_v[r, s])
                    plsc.addupdate(bufs[b].at[r + SEQLEN, s], pe_v[r, s])
            pl.loop(0, SEQLEN, unroll=2)(pe_body)

            pltpu.sync_copy(
                bufs[b],
                out_hbm.at[pl.ds(base + chunk_i * CHUNK, CHUNK)])

        issue_idx(0, 0)
        issue_gathers(0, 0)
        issue_idx(1, 1)

        def body(ii):
            for b in range(2):
                chunk_i = ii + b
                issue_gathers(chunk_i + 1, 1 - b)
                finish_chunk(chunk_i, b)
                issue_idx(chunk_i + 2, b)
        pl.loop(0, NCHUNK - 2, step=2)(body)

        issue_gathers(NCHUNK - 1, 1)
        finish_chunk(NCHUNK - 2, 0)
        finish_chunk(NCHUNK - 1, 1)

    return k


def kernel(text, table):
    idx = text.reshape(ROWS).astype(jnp.int32)
    pe = _sinusoidal_pe(SEQLEN, HDIM)
    table_pad = jnp.pad(table, ((0, 0), (0, 128 - HDIM)))
    emb128 = _make_sc_kernel()(idx, pe, table_pad)
    out_t = _transpose_out(emb128)
    return jnp.transpose(out_t, (2, 0, 1))


# speedup vs baseline: 1.5232x; 1.0952x over previous
import functools
import math

import jax
import jax.numpy as jnp
from jax import lax
from jax.experimental import pallas as pl
from jax.experimental.pallas import tpu as pltpu
from jax.experimental.pallas import tpu_sc as plsc

VOCAB = 1000000
HDIM = 64
BATCH = 4096
SEQLEN = 200

NUM_WORKERS = 32
ROWS = BATCH * SEQLEN
ROWS_PER_WORKER = ROWS // NUM_WORKERS
CHUNK = SEQLEN
NCHUNK = ROWS_PER_WORKER // CHUNK

GATHER_SPLITS = ((0, 104), (104, 96))

BB = 128


def _sinusoidal_pe(length, d_model):
    pos = jnp.arange(length, dtype=jnp.float32)[:, None]
    i = jnp.arange(0, d_model, 2, dtype=jnp.float32)
    div = jnp.exp(-(math.log(10000.0)) * i / d_model)
    pe = jnp.zeros((length, d_model), dtype=jnp.float32)
    pe = pe.at[:, 0::2].set(jnp.sin(pos * div))
    pe = pe.at[:, 1::2].set(jnp.cos(pos * div))
    return pe


def _transpose_out(emb3):
    def body(in_ref, out_ref):
        for j in range(SEQLEN):
            half = (j % 2) * HDIM
            out_ref[j] = in_ref[:, j // 2, half:half + HDIM].T

    return pl.pallas_call(
        body,
        grid=(BATCH // BB,),
        in_specs=[pl.BlockSpec((BB, SEQLEN // 2, 128), lambda b: (b, 0, 0))],
        out_specs=pl.BlockSpec((SEQLEN, HDIM, BB), lambda b: (0, 0, b)),
        out_shape=jax.ShapeDtypeStruct((SEQLEN, HDIM, BATCH), jnp.float32),
    )(emb3)


def _make_sc_kernel():
    mesh = plsc.VectorSubcoreMesh(core_axis_name="c", subcore_axis_name="s",
                                  num_cores=2, num_subcores=16)

    @functools.partial(
        pl.kernel,
        mesh=mesh,
        out_type=jax.ShapeDtypeStruct((ROWS, HDIM), jnp.float32),
        scratch_types=[
            pltpu.VMEM((ROWS_PER_WORKER,), jnp.int32),
            pltpu.VMEM((CHUNK, HDIM), jnp.float32),
            pltpu.VMEM((CHUNK, HDIM), jnp.float32),
            pltpu.VMEM((SEQLEN, HDIM), jnp.float32),
            pltpu.SemaphoreType.DMA,
            pltpu.SemaphoreType.DMA,
        ],
        compiler_params=pltpu.CompilerParams(use_tc_tiling_on_sc=False),
    )
    def k(idx_hbm, pe_hbm, table_hbm, out_hbm,
          idx_v, buf0, buf1, pe_v, gsem0, gsem1):
        wid = lax.axis_index("s") * 2 + lax.axis_index("c")
        base = wid * ROWS_PER_WORKER
        bufs = (buf0, buf1)
        gsems = (gsem0, gsem1)

        pltpu.sync_copy(pe_hbm, pe_v)
        pltpu.sync_copy(idx_hbm.at[pl.ds(base, ROWS_PER_WORKER)], idx_v)

        def issue_gathers(chunk_i, b):
            for off, width in GATHER_SPLITS:
                pltpu.async_copy(
                    table_hbm.at[idx_v.at[pl.ds(chunk_i * CHUNK + off,
                                                width)]],
                    bufs[b].at[pl.ds(off, width)],
                    gsems[b],
                )

        def finish_chunk(chunk_i, b):
            for off, width in GATHER_SPLITS:
                pltpu.make_async_copy(
                    table_hbm.at[idx_v.at[pl.ds(off, width)]],
                    bufs[b].at[pl.ds(off, width)],
                    gsems[b],
                ).wait()

            def pe_body(r):
                for c in range(HDIM // 16):
                    plsc.addupdate(bufs[b].at[r, pl.ds(c * 16, 16)],
                                   pe_v[r, pl.ds(c * 16, 16)])
            pl.loop(0, CHUNK, unroll=4)(pe_body)

            pltpu.sync_copy(bufs[b],
                            out_hbm.at[pl.ds(base + chunk_i * CHUNK, CHUNK)])

        issue_gathers(0, 0)

        def body(ii):
            for b in range(2):
                chunk_i = ii + b
                issue_gathers(chunk_i + 1, 1 - b)
                finish_chunk(chunk_i, b)
        pl.loop(0, NCHUNK - 2, step=2)(body)

        issue_gathers(NCHUNK - 1, 1)
        finish_chunk(NCHUNK - 2, 0)
        finish_chunk(NCHUNK - 1, 1)

    return k


def kernel(text, table):
    idx = text.reshape(ROWS).astype(jnp.int32)
    pe = _sinusoidal_pe(SEQLEN, HDIM)
    emb = _make_sc_kernel()(idx, pe, table)
    out_t = _transpose_out(emb.reshape(BATCH, SEQLEN // 2, 128))
    return jnp.transpose(out_t, (2, 0, 1))
